# baseline (speedup 1.0000x reference)
import functools

import jax
import jax.numpy as jnp
from jax import lax
from jax.experimental import pallas as pl
from jax.experimental.pallas import tpu as pltpu
from jax.experimental.pallas import tpu_sc as plsc

N = 10000
E = 320000
D = 128
G = 128
EPS = 1e-5

NC = 2
NS = 16
NW = NC * NS
SCHUNK = 256
EPW = 10240
EPAD = NW * EPW
NPAD = 10240
RPT = NPAD // NS


def _sc_agg_body(h_hbm, src_hbm, dst_hbm, z128_hbm, agg_out,
                 src_v, dst_v, rows_v, acc_sh, sem):
    c = lax.axis_index("c")
    s = lax.axis_index("s")
    wid = s * NC + c
    pltpu.sync_copy(z128_hbm.at[pl.ds(s * RPT, RPT)], acc_sh.at[pl.ds(s * RPT, RPT)])
    plsc.subcore_barrier()

    for half in range(2):
        base = wid * EPW + half * (EPW // 2)
        pltpu.sync_copy(src_hbm.at[pl.ds(base, EPW // 2)], src_v)
        pltpu.sync_copy(dst_hbm.at[pl.ds(base, EPW // 2)], dst_v)

        def step(j, carry):
            idx = pl.ds(j * SCHUNK, SCHUNK)
            pltpu.async_copy(h_hbm.at[src_v.at[idx]], rows_v, sem).wait()
            pltpu.sync_copy(rows_v, acc_sh.at[dst_v.at[idx]], add=True)
            return carry

        lax.fori_loop(0, EPW // 2 // SCHUNK, step, 0)
    plsc.subcore_barrier()
    pltpu.sync_copy(acc_sh.at[pl.ds(s * RPT, RPT)], agg_out.at[c].at[pl.ds(s * RPT, RPT)])


def _sc_cnt_body(dst_hbm, cnt_out, dst_v, hist_v):
    c = lax.axis_index("c")
    s = lax.axis_index("s")
    wid = s * NC + c
    zeros16 = jnp.zeros((16,), jnp.float32)
    ones16 = jnp.ones((16,), jnp.float32)

    def zstep(i, carry):
        hist_v[pl.ds(i * 16, 16)] = zeros16
        return carry

    lax.fori_loop(0, NPAD // 16, zstep, 0)

    for half in range(2):
        base = wid * EPW + half * (EPW // 2)
        pltpu.sync_copy(dst_hbm.at[pl.ds(base, EPW // 2)], dst_v)

        def step(i, carry):
            idx = dst_v[pl.ds(i * 16, 16)]
            plsc.addupdate_scatter(hist_v, [idx], ones16)
            return carry

        lax.fori_loop(0, EPW // 2 // 16, step, 0)
    pltpu.sync_copy(hist_v, cnt_out.at[pl.ds(wid * NPAD, NPAD)])


@functools.lru_cache(maxsize=None)
def _get_sc_kernels():
    mesh = plsc.VectorSubcoreMesh(core_axis_name="c", subcore_axis_name="s")
    sc_agg = pl.kernel(
        _sc_agg_body,
        out_type=jax.ShapeDtypeStruct((NC, NPAD, D), jnp.float32),
        mesh=mesh,
        scratch_types=[
            pltpu.VMEM((EPW // 2,), jnp.int32),
            pltpu.VMEM((EPW // 2,), jnp.int32),
            pltpu.VMEM((SCHUNK, D), jnp.float32),
            pltpu.VMEM_SHARED((NPAD, D), jnp.float32),
            pltpu.SemaphoreType.DMA,
        ],
    )
    sc_cnt = pl.kernel(
        _sc_cnt_body,
        out_type=jax.ShapeDtypeStruct((NW * NPAD,), jnp.float32),
        mesh=mesh,
        scratch_types=[
            pltpu.VMEM((EPW // 2,), jnp.int32),
            pltpu.VMEM((NPAD,), jnp.float32),
        ],
        compiler_params=pltpu.CompilerParams(needs_layout_passes=False),
    )
    return sc_agg, sc_cnt


def _tc_layer_first_body(agg_ref, cnt_ref, h_ref, wl_ref, wr_ref, b_ref, g_ref,
                         be_ref, out_ref, inv_ref):
    cnt = jnp.sum(cnt_ref[...], axis=0)
    inv = 1.0 / jnp.maximum(cnt, 1.0)
    inv_ref[...] = inv
    agg3 = (agg_ref[0] + agg_ref[1]).reshape(NPAD // 128, 128, D) * inv[:, :, None]
    agg = agg3.reshape(NPAD, D)[:N]
    z = (jnp.dot(agg, wl_ref[...], preferred_element_type=jnp.float32)
         + jnp.dot(h_ref[...], wr_ref[...], preferred_element_type=jnp.float32)
         + b_ref[...])
    z = jnp.maximum(z, 0.0)
    mu = jnp.mean(z, axis=0, keepdims=True)
    var = jnp.mean((z - mu) ** 2, axis=0, keepdims=True)
    out_ref[...] = (z - mu) * jax.lax.rsqrt(var + EPS) * g_ref[...] + be_ref[...]


def _tc_layer_body(agg_ref, inv_ref, h_ref, wl_ref, wr_ref, b_ref, g_ref,
                   be_ref, out_ref):
    agg3 = (agg_ref[0] + agg_ref[1]).reshape(NPAD // 128, 128, D) * inv_ref[...][:, :, None]
    agg = agg3.reshape(NPAD, D)[:N]
    z = (jnp.dot(agg, wl_ref[...], preferred_element_type=jnp.float32)
         + jnp.dot(h_ref[...], wr_ref[...], preferred_element_type=jnp.float32)
         + b_ref[...])
    z = jnp.maximum(z, 0.0)
    mu = jnp.mean(z, axis=0, keepdims=True)
    var = jnp.mean((z - mu) ** 2, axis=0, keepdims=True)
    out_ref[...] = (z - mu) * jax.lax.rsqrt(var + EPS) * g_ref[...] + be_ref[...]


def _tc_final_body(agg_ref, inv_ref, h_ref, wl_ref, wr_ref, b_ref, g_ref,
                   be_ref, batch_ref, fcw_ref, fcb_ref, out_ref):
    agg3 = (agg_ref[0] + agg_ref[1]).reshape(NPAD // 128, 128, D) * inv_ref[...][:, :, None]
    agg = agg3.reshape(NPAD, D)[:N]
    z = (jnp.dot(agg, wl_ref[...], preferred_element_type=jnp.float32)
         + jnp.dot(h_ref[...], wr_ref[...], preferred_element_type=jnp.float32)
         + b_ref[...])
    z = jnp.maximum(z, 0.0)
    mu = jnp.mean(z, axis=0, keepdims=True)
    var = jnp.mean((z - mu) ** 2, axis=0, keepdims=True)
    h = (z - mu) * jax.lax.rsqrt(var + EPS) * g_ref[...] + be_ref[...]
    gids = jax.lax.broadcasted_iota(jnp.int32, (G, N), 0)
    mask = (gids == batch_ref[...]).astype(jnp.float32)
    pooled = jnp.dot(mask, h, preferred_element_type=jnp.float32)
    logit = jnp.dot(pooled, fcw_ref[...], preferred_element_type=jnp.float32) + fcb_ref[...]
    out_ref[...] = 1.0 / (1.0 + jnp.exp(-logit))


_tc_layer_first = pl.pallas_call(
    _tc_layer_first_body,
    out_shape=(jax.ShapeDtypeStruct((N, D), jnp.float32),
               jax.ShapeDtypeStruct((NPAD // 128, 128), jnp.float32)),
)

_tc_layer = pl.pallas_call(
    _tc_layer_body,
    out_shape=jax.ShapeDtypeStruct((N, D), jnp.float32),
)

_tc_final = pl.pallas_call(
    _tc_final_body,
    out_shape=jax.ShapeDtypeStruct((G, 1), jnp.float32),
)


def kernel(x, edge_index, edge_attr, batch, Wl0, Wr0, b0, g0, be0, Wl1, Wr1,
           b1, g1, be1, Wl2, Wr2, b2, g2, be2, fcW, fcb):
    pad = EPAD - edge_index.shape[1]
    src3 = jnp.concatenate([edge_index[0], jnp.zeros((pad,), jnp.int32)])
    pad_dst = N + jnp.arange(pad, dtype=jnp.int32) % (NPAD - N)
    dst3 = jnp.concatenate([edge_index[1], pad_dst])
    z128 = jnp.zeros((NPAD, D), jnp.float32)
    sc_agg, sc_cnt = _get_sc_kernels()
    cntp = sc_cnt(dst3).reshape(NW, NPAD // 128, 128)
    aggp = sc_agg(x, src3, dst3, z128)
    h1, inv = _tc_layer_first(aggp, cntp, x, Wl0, Wr0, b0.reshape(1, D),
                              g0.reshape(1, D), be0.reshape(1, D))
    aggp = sc_agg(h1, src3, dst3, z128)
    h2 = _tc_layer(aggp, inv, h1, Wl1, Wr1, b1.reshape(1, D),
                   g1.reshape(1, D), be1.reshape(1, D))
    aggp = sc_agg(h2, src3, dst3, z128)
    out = _tc_final(aggp, inv, h2, Wl2, Wr2, b2.reshape(1, D),
                    g2.reshape(1, D), be2.reshape(1, D),
                    batch.reshape(1, N), fcW, fcb.reshape(1, 1))
    return out


# speedup vs baseline: 1.0460x; 1.0460x over previous
import functools

import jax
import jax.numpy as jnp
from jax import lax
from jax.experimental import pallas as pl
from jax.experimental.pallas import tpu as pltpu
from jax.experimental.pallas import tpu_sc as plsc

N = 10000
E = 320000
D = 128
G = 128
EPS = 1e-5

NC = 2
NS = 16
NW = NC * NS
CHUNK = 128
NCH = 80
HCH = NCH // 2
EPW = NCH * CHUNK
EPAD = NW * EPW
NPAD = 10240
RPT = NPAD // NS


def _sc_agg_body(h_hbm, src_hbm, dst_hbm, z128_hbm, agg_out,
                 src_v, dst_v, rows_a, rows_b, acc_sh, sem_a, sem_b):
    c = lax.axis_index("c")
    s = lax.axis_index("s")
    wid = s * NC + c
    pltpu.sync_copy(z128_hbm.at[pl.ds(s * RPT, RPT)], acc_sh.at[pl.ds(s * RPT, RPT)])
    plsc.subcore_barrier()

    for half in range(2):
        pltpu.sync_copy(src_hbm.at[wid].at[half], src_v)
        pltpu.sync_copy(dst_hbm.at[wid].at[half], dst_v)

        pltpu.async_copy(h_hbm.at[src_v.at[0]], rows_a, sem_a)

        def step(i, carry):
            pltpu.make_async_copy(h_hbm.at[src_v.at[0]], rows_a, sem_a).wait()
            pltpu.async_copy(h_hbm.at[src_v.at[2 * i + 1]], rows_b, sem_b)
            pltpu.sync_copy(rows_a, acc_sh.at[dst_v.at[2 * i]], add=True)
            pltpu.make_async_copy(h_hbm.at[src_v.at[0]], rows_b, sem_b).wait()

            @pl.when(i < HCH // 2 - 1)
            def _():
                pltpu.async_copy(h_hbm.at[src_v.at[2 * i + 2]], rows_a, sem_a)

            pltpu.sync_copy(rows_b, acc_sh.at[dst_v.at[2 * i + 1]], add=True)
            return carry

        lax.fori_loop(0, HCH // 2, step, 0)
    plsc.subcore_barrier()
    pltpu.sync_copy(acc_sh.at[pl.ds(s * RPT, RPT)], agg_out.at[c].at[pl.ds(s * RPT, RPT)])


def _sc_cnt_body(dst_hbm, cnt_out, dst_v, hist_v):
    c = lax.axis_index("c")
    s = lax.axis_index("s")
    wid = s * NC + c
    zeros16 = jnp.zeros((16,), jnp.float32)
    ones16 = jnp.ones((16,), jnp.float32)

    def zstep(i, carry):
        hist_v[pl.ds(i * 16, 16)] = zeros16
        return carry

    lax.fori_loop(0, NPAD // 16, zstep, 0)

    for half in range(2):
        pltpu.sync_copy(dst_hbm.at[wid].at[half], dst_v)

        def step(i, carry):
            idx = dst_v[i >> 3, pl.ds((i & 7) * 16, 16)]
            plsc.addupdate_scatter(hist_v, [idx], ones16)
            return carry

        lax.fori_loop(0, HCH * CHUNK // 16, step, 0)
    pltpu.sync_copy(hist_v, cnt_out.at[pl.ds(wid * NPAD, NPAD)])


@functools.lru_cache(maxsize=None)
def _get_sc_kernels():
    mesh = plsc.VectorSubcoreMesh(core_axis_name="c", subcore_axis_name="s")
    sc_agg = pl.kernel(
        _sc_agg_body,
        out_type=jax.ShapeDtypeStruct((NC, NPAD, D), jnp.float32),
        mesh=mesh,
        scratch_types=[
            pltpu.VMEM((HCH, CHUNK), jnp.int32),
            pltpu.VMEM((HCH, CHUNK), jnp.int32),
            pltpu.VMEM((CHUNK, D), jnp.float32),
            pltpu.VMEM((CHUNK, D), jnp.float32),
            pltpu.VMEM_SHARED((NPAD, D), jnp.float32),
            pltpu.SemaphoreType.DMA,
            pltpu.SemaphoreType.DMA,
        ],
    )
    sc_cnt = pl.kernel(
        _sc_cnt_body,
        out_type=jax.ShapeDtypeStruct((NW * NPAD,), jnp.float32),
        mesh=mesh,
        scratch_types=[
            pltpu.VMEM((HCH, CHUNK), jnp.int32),
            pltpu.VMEM((NPAD,), jnp.float32),
        ],
        compiler_params=pltpu.CompilerParams(needs_layout_passes=False),
    )
    return sc_agg, sc_cnt


def _tc_layer_first_body(agg_ref, cnt_ref, h_ref, wl_ref, wr_ref, b_ref, g_ref,
                         be_ref, out_ref, inv_ref):
    cnt = jnp.sum(cnt_ref[...], axis=0)
    inv = 1.0 / jnp.maximum(cnt, 1.0)
    inv_ref[...] = inv
    agg3 = (agg_ref[0] + agg_ref[1]).reshape(NPAD // 128, 128, D) * inv[:, :, None]
    agg = agg3.reshape(NPAD, D)[:N]
    z = (jnp.dot(agg, wl_ref[...], preferred_element_type=jnp.float32)
         + jnp.dot(h_ref[...], wr_ref[...], preferred_element_type=jnp.float32)
         + b_ref[...])
    z = jnp.maximum(z, 0.0)
    mu = jnp.mean(z, axis=0, keepdims=True)
    var = jnp.mean((z - mu) ** 2, axis=0, keepdims=True)
    out_ref[...] = (z - mu) * jax.lax.rsqrt(var + EPS) * g_ref[...] + be_ref[...]


def _tc_layer_body(agg_ref, inv_ref, h_ref, wl_ref, wr_ref, b_ref, g_ref,
                   be_ref, out_ref):
    agg3 = (agg_ref[0] + agg_ref[1]).reshape(NPAD // 128, 128, D) * inv_ref[...][:, :, None]
    agg = agg3.reshape(NPAD, D)[:N]
    z = (jnp.dot(agg, wl_ref[...], preferred_element_type=jnp.float32)
         + jnp.dot(h_ref[...], wr_ref[...], preferred_element_type=jnp.float32)
         + b_ref[...])
    z = jnp.maximum(z, 0.0)
    mu = jnp.mean(z, axis=0, keepdims=True)
    var = jnp.mean((z - mu) ** 2, axis=0, keepdims=True)
    out_ref[...] = (z - mu) * jax.lax.rsqrt(var + EPS) * g_ref[...] + be_ref[...]


def _tc_final_body(agg_ref, inv_ref, h_ref, wl_ref, wr_ref, b_ref, g_ref,
                   be_ref, batch_ref, fcw_ref, fcb_ref, out_ref):
    agg3 = (agg_ref[0] + agg_ref[1]).reshape(NPAD // 128, 128, D) * inv_ref[...][:, :, None]
    agg = agg3.reshape(NPAD, D)[:N]
    z = (jnp.dot(agg, wl_ref[...], preferred_element_type=jnp.float32)
         + jnp.dot(h_ref[...], wr_ref[...], preferred_element_type=jnp.float32)
         + b_ref[...])
    z = jnp.maximum(z, 0.0)
    mu = jnp.mean(z, axis=0, keepdims=True)
    var = jnp.mean((z - mu) ** 2, axis=0, keepdims=True)
    h = (z - mu) * jax.lax.rsqrt(var + EPS) * g_ref[...] + be_ref[...]
    gids = jax.lax.broadcasted_iota(jnp.int32, (G, N), 0)
    mask = (gids == batch_ref[...]).astype(jnp.float32)
    pooled = jnp.dot(mask, h, preferred_element_type=jnp.float32)
    logit = jnp.dot(pooled, fcw_ref[...], preferred_element_type=jnp.float32) + fcb_ref[...]
    out_ref[...] = 1.0 / (1.0 + jnp.exp(-logit))


_tc_layer_first = pl.pallas_call(
    _tc_layer_first_body,
    out_shape=(jax.ShapeDtypeStruct((N, D), jnp.float32),
               jax.ShapeDtypeStruct((NPAD // 128, 128), jnp.float32)),
)

_tc_layer = pl.pallas_call(
    _tc_layer_body,
    out_shape=jax.ShapeDtypeStruct((N, D), jnp.float32),
)

_tc_final = pl.pallas_call(
    _tc_final_body,
    out_shape=jax.ShapeDtypeStruct((G, 1), jnp.float32),
)


def kernel(x, edge_index, edge_attr, batch, Wl0, Wr0, b0, g0, be0, Wl1, Wr1,
           b1, g1, be1, Wl2, Wr2, b2, g2, be2, fcW, fcb):
    pad = EPAD - edge_index.shape[1]
    src3 = jnp.concatenate(
        [edge_index[0], jnp.zeros((pad,), jnp.int32)]).reshape(NW, 2, HCH, CHUNK)
    pad_dst = N + jnp.arange(pad, dtype=jnp.int32) % (NPAD - N)
    dst3 = jnp.concatenate([edge_index[1], pad_dst]).reshape(NW, 2, HCH, CHUNK)
    z128 = jnp.zeros((NPAD, D), jnp.float32)
    sc_agg, sc_cnt = _get_sc_kernels()
    cntp = sc_cnt(dst3).reshape(NW, NPAD // 128, 128)
    aggp = sc_agg(x, src3, dst3, z128)
    h1, inv = _tc_layer_first(aggp, cntp, x, Wl0, Wr0, b0.reshape(1, D),
                              g0.reshape(1, D), be0.reshape(1, D))
    aggp = sc_agg(h1, src3, dst3, z128)
    h2 = _tc_layer(aggp, inv, h1, Wl1, Wr1, b1.reshape(1, D),
                   g1.reshape(1, D), be1.reshape(1, D))
    aggp = sc_agg(h2, src3, dst3, z128)
    out = _tc_final(aggp, inv, h2, Wl2, Wr2, b2.reshape(1, D),
                    g2.reshape(1, D), be2.reshape(1, D),
                    batch.reshape(1, N), fcW, fcb.reshape(1, 1))
    return out
